# baseline (speedup 1.0000x reference)
import functools

import jax
import jax.numpy as jnp
from jax import lax
from jax.experimental import pallas as pl
from jax.experimental.pallas import tpu as pltpu
from jax.experimental.pallas import tpu_sc as plsc

N = 100000
E = 6400000
LANES = 128
ROWS = 784
NP = ROWS * LANES
NC = 2
NS = 16
NW = NC * NS
PT = NP // NS
EW = E // NW
CA = 10000
CB = 1600
ZCH = 1568
L = 16
RING = 4

_mesh = plsc.VectorSubcoreMesh(core_axis_name="c", subcore_axis_name="s")
_sc_params = pltpu.CompilerParams(needs_layout_passes=False)


def _zero_shared(buf, acc_sh, s):
    def fill(i, carry):
        buf[pl.ds(i * L, L)] = jnp.zeros((L,), jnp.float32)
        return carry

    lax.fori_loop(0, ZCH // L, fill, 0)
    for q in range(4):
        pltpu.sync_copy(buf.at[pl.ds(0, ZCH)],
                        acc_sh.at[pl.ds(s * PT + q * ZCH, ZCH)])


def _rsqrt_guarded(d):
    i = plsc.bitcast(d, jnp.int32)
    y = plsc.bitcast(jnp.int32(0x5F3759DF) - (i >> 1), jnp.float32)
    half_d = 0.5 * d
    for _ in range(3):
        y = y * (1.5 - half_d * y * y)
    return jnp.where(d > 0.0, y, 0.0)


@functools.partial(
    pl.kernel,
    out_type=jax.ShapeDtypeStruct((NW * PT,), jnp.float32),
    mesh=_mesh,
    compiler_params=_sc_params,
    scratch_types=[
        [pltpu.VMEM((CA,), jnp.int32) for _ in range(RING)],
        [pltpu.VMEM((CA,), jnp.float32) for _ in range(RING)],
        pltpu.VMEM_SHARED((NP,), jnp.float32),
        [pltpu.SemaphoreType.DMA for _ in range(RING)],
        [pltpu.SemaphoreType.DMA for _ in range(RING)],
    ],
)
def _degree(col_hbm, w_hbm, out, col_b, w_b, acc_sh, in_sems, sc_sems):
    c = lax.axis_index("c")
    s = lax.axis_index("s")
    wid = c * NS + s
    nch = EW // CA

    def in_start(k, r):
        base = wid * EW + k * CA
        pltpu.async_copy(col_hbm.at[pl.ds(base, CA)], col_b[r], in_sems[r])
        pltpu.async_copy(w_hbm.at[pl.ds(base, CA)], w_b[r], in_sems[r])

    def in_wait(r):
        pltpu.make_async_copy(col_hbm.at[pl.ds(0, CA)], col_b[r], in_sems[r]).wait()
        pltpu.make_async_copy(w_hbm.at[pl.ds(0, CA)], w_b[r], in_sems[r]).wait()

    def sc_wait(r):
        pltpu.make_async_copy(w_b[r], acc_sh.at[col_b[r]], sc_sems[r]).wait()

    in_start(0, 0)
    in_start(1, 1)
    _zero_shared(w_b[3], acc_sh, s)
    plsc.subcore_barrier()

    def step(t, carry):
        for u in range(RING):
            k = t * RING + u
            in_wait(u)
            pltpu.async_copy(w_b[u], acc_sh.at[col_b[u]], sc_sems[u], add=True)
            nxt = (u + 2) % RING

            @pl.when(k >= 2)
            def _():
                sc_wait(nxt)

            @pl.when(k + 2 < nch)
            def _():
                in_start(k + 2, nxt)

        return carry

    lax.fori_loop(0, nch // RING, step, 0)
    for k in range(nch - 2, nch):
        sc_wait(k % RING)
    plsc.subcore_barrier()
    pltpu.sync_copy(acc_sh.at[pl.ds(s * PT, PT)],
                    out.at[pl.ds(pl.multiple_of(wid * PT, 8), PT)])


def _edge_loop(row_hbm, col_hbm, w_hbm, out, table_v,
               row_b, col_b, w_b, acc_sh, in_sems, sc_sems, wid, s):
    nch = EW // CB

    def in_start(k, r):
        base = wid * EW + k * CB
        pltpu.async_copy(row_hbm.at[pl.ds(base, CB)], row_b[r], in_sems[r])
        pltpu.async_copy(col_hbm.at[pl.ds(base, CB)], col_b[r], in_sems[r])
        pltpu.async_copy(w_hbm.at[pl.ds(base, CB)], w_b[r], in_sems[r])

    def in_wait(r):
        pltpu.make_async_copy(row_hbm.at[pl.ds(0, CB)], row_b[r], in_sems[r]).wait()
        pltpu.make_async_copy(col_hbm.at[pl.ds(0, CB)], col_b[r], in_sems[r]).wait()
        pltpu.make_async_copy(w_hbm.at[pl.ds(0, CB)], w_b[r], in_sems[r]).wait()

    def sc_wait(r):
        pltpu.make_async_copy(w_b[r], acc_sh.at[col_b[r]], sc_sems[r]).wait()

    in_start(0, 0)
    in_start(1, 1)

    def do_chunk(k, u):
        in_wait(u)

        @plsc.parallel_loop(0, CB // L, unroll=5)
        def grp(j):
            sl = pl.ds(j * L, L)
            g = plsc.load_gather(table_v, [row_b[u][sl]])
            w_b[u][sl] = g * w_b[u][sl]

        pltpu.async_copy(w_b[u], acc_sh.at[col_b[u]], sc_sems[u], add=True)
        nxt = (u + 2) % RING

        if isinstance(k, int):
            if k >= 2:
                sc_wait(nxt)
            if k + 2 < nch:
                in_start(k + 2, nxt)
        else:
            @pl.when(k >= 2)
            def _():
                sc_wait(nxt)

            @pl.when(k + 2 < nch)
            def _():
                in_start(k + 2, nxt)

    def step(t, carry):
        for u in range(RING):
            do_chunk(t * RING + u, u)
        return carry

    full = (nch // RING) * RING
    lax.fori_loop(0, nch // RING, step, 0)
    for k in range(full, nch):
        do_chunk(k, k % RING)
    for k in range(nch - 2, nch):
        sc_wait(k % RING)
    plsc.subcore_barrier()
    pltpu.sync_copy(acc_sh.at[pl.ds(s * PT, PT)],
                    out.at[pl.ds(pl.multiple_of(wid * PT, 8), PT)])


_prop_scratch = [
    pltpu.VMEM((NP,), jnp.float32),
    [pltpu.VMEM((CB,), jnp.int32) for _ in range(RING)],
    [pltpu.VMEM((CB,), jnp.int32) for _ in range(RING)],
    [pltpu.VMEM((CB,), jnp.float32) for _ in range(RING)],
    pltpu.VMEM((L,), jnp.float32),
    pltpu.VMEM_SHARED((NP,), jnp.float32),
    [pltpu.SemaphoreType.DMA for _ in range(RING)],
    [pltpu.SemaphoreType.DMA for _ in range(RING)],
]


@functools.partial(
    pl.kernel,
    out_type=(
        jax.ShapeDtypeStruct((NW * PT,), jnp.float32),
        jax.ShapeDtypeStruct((NC * NP,), jnp.float32),
        jax.ShapeDtypeStruct((NP,), jnp.float32),
        jax.ShapeDtypeStruct((NP,), jnp.float32),
    ),
    mesh=_mesh,
    compiler_params=_sc_params,
    scratch_types=_prop_scratch,
)
def _prop_b(row_hbm, col_hbm, w_hbm, degp_hbm, x0_hbm, x1_hbm, wv_hbm,
            t_out, tstage, dinv_out, ua1_out,
            table_v, row_b, col_b, w_b, wv_v, acc_sh, in_sems, sc_sems):
    c = lax.axis_index("c")
    s = lax.axis_index("s")
    wid = c * NS + s

    pltpu.sync_copy(wv_hbm, wv_v)
    wvec = wv_v[pl.ds(0, L)]
    w2 = wvec[2]
    w3 = wvec[3]
    w4 = wvec[4]
    w5 = wvec[5]

    for q in range(4):
        nbase = pl.multiple_of(s * PT + q * ZCH, 8)
        pltpu.sync_copy(degp_hbm.at[pl.ds(nbase, ZCH)],
                        w_b[0].at[pl.ds(0, ZCH)])
        pltpu.sync_copy(degp_hbm.at[pl.ds(pl.multiple_of(NP + nbase, 8), ZCH)],
                        w_b[1].at[pl.ds(0, ZCH)])
        pltpu.sync_copy(x0_hbm.at[pl.ds(nbase, ZCH)], w_b[2].at[pl.ds(0, ZCH)])
        pltpu.sync_copy(x1_hbm.at[pl.ds(nbase, ZCH)], w_b[3].at[pl.ds(0, ZCH)])

        def tgrp(j, carry):
            sl = pl.ds(j * L, L)
            d = w_b[0][sl] + w_b[1][sl]
            dinv = _rsqrt_guarded(d)
            xx0 = w_b[2][sl]
            xx1 = w_b[3][sl]
            w_b[0][sl] = dinv * (xx0 * w4 + xx1 * w5)
            w_b[1][sl] = dinv * (xx0 * w2 + xx1 * w3)
            w_b[2][sl] = dinv
            return carry

        lax.fori_loop(0, ZCH // L, tgrp, 0)
        pltpu.sync_copy(w_b[0].at[pl.ds(0, ZCH)],
                        tstage.at[pl.ds(pl.multiple_of(c * NP + nbase, 8), ZCH)])

        @pl.when(c == 0)
        def _():
            pltpu.sync_copy(w_b[1].at[pl.ds(0, ZCH)],
                            ua1_out.at[pl.ds(nbase, ZCH)])
            pltpu.sync_copy(w_b[2].at[pl.ds(0, ZCH)],
                            dinv_out.at[pl.ds(nbase, ZCH)])

    _zero_shared(w_b[3], acc_sh, s)
    plsc.subcore_barrier()
    pltpu.sync_copy(tstage.at[pl.ds(pl.multiple_of(c * NP, 8), NP)], table_v)
    _edge_loop(row_hbm, col_hbm, w_hbm, t_out, table_v,
               row_b, col_b, w_b, acc_sh, in_sems, sc_sems, wid, s)


@functools.partial(
    pl.kernel,
    out_type=(
        jax.ShapeDtypeStruct((NW * PT,), jnp.float32),
        jax.ShapeDtypeStruct((NC * NP,), jnp.float32),
    ),
    mesh=_mesh,
    compiler_params=_sc_params,
    scratch_types=_prop_scratch,
)
def _prop_c(row_hbm, col_hbm, w_hbm, tp_hbm, dinv_hbm, ua1_hbm,
            s_out, tstage,
            table_v, row_b, col_b, w_b, wv_v, acc_sh, in_sems, sc_sems):
    c = lax.axis_index("c")
    s = lax.axis_index("s")
    wid = c * NS + s

    for q in range(4):
        nbase = pl.multiple_of(s * PT + q * ZCH, 8)
        pltpu.sync_copy(tp_hbm.at[pl.ds(nbase, ZCH)],
                        w_b[0].at[pl.ds(0, ZCH)])
        pltpu.sync_copy(tp_hbm.at[pl.ds(pl.multiple_of(NP + nbase, 8), ZCH)],
                        w_b[1].at[pl.ds(0, ZCH)])
        pltpu.sync_copy(dinv_hbm.at[pl.ds(nbase, ZCH)], w_b[2].at[pl.ds(0, ZCH)])
        pltpu.sync_copy(ua1_hbm.at[pl.ds(nbase, ZCH)], w_b[3].at[pl.ds(0, ZCH)])

        def tgrp(j, carry):
            sl = pl.ds(j * L, L)
            dv = w_b[2][sl]
            w_b[0][sl] = w_b[3][sl] + dv * dv * (w_b[0][sl] + w_b[1][sl])
            return carry

        lax.fori_loop(0, ZCH // L, tgrp, 0)
        pltpu.sync_copy(w_b[0].at[pl.ds(0, ZCH)],
                        tstage.at[pl.ds(pl.multiple_of(c * NP + nbase, 8), ZCH)])

    _zero_shared(w_b[3], acc_sh, s)
    plsc.subcore_barrier()
    pltpu.sync_copy(tstage.at[pl.ds(pl.multiple_of(c * NP, 8), NP)], table_v)
    _edge_loop(row_hbm, col_hbm, w_hbm, s_out, table_v,
               row_b, col_b, w_b, acc_sh, in_sems, sc_sems, wid, s)


def _stage3_body(s0, s1, dinv, x0, x1, wv, out_o):
    a0 = x0[...] * wv[0] + x1[...] * wv[1]
    out_o[...] = jnp.maximum(
        a0 + dinv[...] * (s0[...] + s1[...]) + wv[6], 0.0)


_vspec = pl.BlockSpec(memory_space=pltpu.VMEM)
_sspec = pl.BlockSpec(memory_space=pltpu.SMEM)
_nshape = jax.ShapeDtypeStruct((NP,), jnp.float32)

_stage3 = pl.pallas_call(
    _stage3_body,
    out_shape=_nshape,
    in_specs=[_vspec, _vspec, _vspec, _vspec, _vspec, _sspec],
    out_specs=_vspec,
)


def kernel(x, edge_index, edge_weight, W0, W1, W2, b):
    row = edge_index[0]
    col = edge_index[1]

    pad = NP - N
    x0 = jnp.pad(x[:, 0], (0, pad))
    x1 = jnp.pad(x[:, 1], (0, pad))
    wv = jnp.stack([W0[0, 0], W0[1, 0], W1[0, 0], W1[1, 0],
                    W2[0, 0], W2[1, 0], b[0], b[0]] * 2)

    deg_parts = _degree(col, edge_weight)

    t_parts, _, dinv, ua1 = _prop_b(row, col, edge_weight,
                                    deg_parts, x0, x1, wv)

    s_parts, _ = _prop_c(row, col, edge_weight, t_parts, dinv, ua1)

    out = _stage3(s_parts[:NP], s_parts[NP:], dinv, x0, x1, wv)

    return out.reshape(NP, 1)[:N]


# speedup vs baseline: 1.1178x; 1.1178x over previous
import functools

import jax
import jax.numpy as jnp
from jax import lax
from jax.experimental import pallas as pl
from jax.experimental.pallas import tpu as pltpu
from jax.experimental.pallas import tpu_sc as plsc

N = 100000
E = 6400000
LANES = 128
ROWS = 784
NP = ROWS * LANES
NC = 2
NS = 16
NW = NC * NS
PT = NP // NS
EW = E // NW
CA = 10000
CB = 1600
ZCH = 1568
L = 16
RING = 4

_mesh = plsc.VectorSubcoreMesh(core_axis_name="c", subcore_axis_name="s")
_sc_params = pltpu.CompilerParams(needs_layout_passes=False)


def _zero_shared(buf, acc_sh, s):
    def fill(i, carry):
        buf[pl.ds(i * L, L)] = jnp.zeros((L,), jnp.float32)
        return carry

    lax.fori_loop(0, ZCH // L, fill, 0)
    for q in range(4):
        pltpu.sync_copy(buf.at[pl.ds(0, ZCH)],
                        acc_sh.at[pl.ds(s * PT + q * ZCH, ZCH)])


@functools.partial(
    pl.kernel,
    out_type=jax.ShapeDtypeStruct((NW, PT), jnp.float32),
    mesh=_mesh,
    compiler_params=_sc_params,
    scratch_types=[
        [pltpu.VMEM((CA,), jnp.int32) for _ in range(RING)],
        [pltpu.VMEM((CA,), jnp.float32) for _ in range(RING)],
        pltpu.VMEM_SHARED((NP,), jnp.float32),
        [pltpu.SemaphoreType.DMA for _ in range(RING)],
        [pltpu.SemaphoreType.DMA for _ in range(RING)],
    ],
)
def _degree(ei_hbm, w_hbm, out, col_b, w_b, acc_sh, in_sems, sc_sems):
    c = lax.axis_index("c")
    s = lax.axis_index("s")
    wid = c * NS + s
    nch = EW // CA

    def in_start(k, r):
        base = wid * EW + k * CA
        pltpu.async_copy(ei_hbm.at[pl.ds(pl.multiple_of(E + base, 8), CA)],
                         col_b[r], in_sems[r])
        pltpu.async_copy(w_hbm.at[pl.ds(base, CA)], w_b[r], in_sems[r])

    def in_wait(r):
        pltpu.make_async_copy(ei_hbm.at[pl.ds(0, CA)], col_b[r], in_sems[r]).wait()
        pltpu.make_async_copy(w_hbm.at[pl.ds(0, CA)], w_b[r], in_sems[r]).wait()

    def sc_wait(r):
        pltpu.make_async_copy(w_b[r], acc_sh.at[col_b[r]], sc_sems[r]).wait()

    in_start(0, 0)
    in_start(1, 1)
    _zero_shared(w_b[3], acc_sh, s)
    plsc.subcore_barrier()

    def step(t, carry):
        for u in range(RING):
            k = t * RING + u
            in_wait(u)
            pltpu.async_copy(w_b[u], acc_sh.at[col_b[u]], sc_sems[u], add=True)
            nxt = (u + 2) % RING

            @pl.when(k >= 2)
            def _():
                sc_wait(nxt)

            @pl.when(k + 2 < nch)
            def _():
                in_start(k + 2, nxt)

        return carry

    lax.fori_loop(0, nch // RING, step, 0)
    for k in range(nch - 2, nch):
        sc_wait(k % RING)
    plsc.subcore_barrier()
    pltpu.sync_copy(acc_sh.at[pl.ds(s * PT, PT)], out.at[wid])


@functools.partial(
    pl.kernel,
    out_type=jax.ShapeDtypeStruct((NW, PT), jnp.float32),
    mesh=_mesh,
    compiler_params=_sc_params,
    scratch_types=[
        pltpu.VMEM((NP,), jnp.float32),
        [pltpu.VMEM((CB,), jnp.int32) for _ in range(RING)],
        [pltpu.VMEM((CB,), jnp.int32) for _ in range(RING)],
        [pltpu.VMEM((CB,), jnp.float32) for _ in range(RING)],
        pltpu.VMEM_SHARED((NP,), jnp.float32),
        [pltpu.SemaphoreType.DMA for _ in range(RING)],
        [pltpu.SemaphoreType.DMA for _ in range(RING)],
    ],
)
def _prop(ei_hbm, w_hbm, table_hbm, out,
          table_v, row_b, col_b, w_b, acc_sh, in_sems, sc_sems):
    c = lax.axis_index("c")
    s = lax.axis_index("s")
    wid = c * NS + s
    nch = EW // CB

    def in_start(k, r):
        base = wid * EW + k * CB
        pltpu.async_copy(ei_hbm.at[pl.ds(pl.multiple_of(base, 8), CB)],
                         row_b[r], in_sems[r])
        pltpu.async_copy(ei_hbm.at[pl.ds(pl.multiple_of(E + base, 8), CB)],
                         col_b[r], in_sems[r])
        pltpu.async_copy(w_hbm.at[pl.ds(base, CB)], w_b[r], in_sems[r])

    def in_wait(r):
        pltpu.make_async_copy(ei_hbm.at[pl.ds(0, CB)], row_b[r], in_sems[r]).wait()
        pltpu.make_async_copy(ei_hbm.at[pl.ds(0, CB)], col_b[r], in_sems[r]).wait()
        pltpu.make_async_copy(w_hbm.at[pl.ds(0, CB)], w_b[r], in_sems[r]).wait()

    def sc_wait(r):
        pltpu.make_async_copy(w_b[r], acc_sh.at[col_b[r]], sc_sems[r]).wait()

    in_start(0, 0)
    in_start(1, 1)
    pltpu.sync_copy(table_hbm, table_v)
    _zero_shared(w_b[3], acc_sh, s)
    plsc.subcore_barrier()

    def do_chunk(k, u):
        in_wait(u)

        @plsc.parallel_loop(0, CB // L, unroll=5)
        def grp(j):
            sl = pl.ds(j * L, L)
            g = plsc.load_gather(table_v, [row_b[u][sl]])
            w_b[u][sl] = g * w_b[u][sl]

        pltpu.async_copy(w_b[u], acc_sh.at[col_b[u]], sc_sems[u], add=True)
        nxt = (u + 2) % RING

        if isinstance(k, int):
            if k >= 2:
                sc_wait(nxt)
            if k + 2 < nch:
                in_start(k + 2, nxt)
        else:
            @pl.when(k >= 2)
            def _():
                sc_wait(nxt)

            @pl.when(k + 2 < nch)
            def _():
                in_start(k + 2, nxt)

    def step(t, carry):
        for u in range(RING):
            do_chunk(t * RING + u, u)
        return carry

    full = (nch // RING) * RING
    lax.fori_loop(0, nch // RING, step, 0)
    for k in range(full, nch):
        do_chunk(k, k % RING)
    for k in range(nch - 2, nch):
        sc_wait(k % RING)
    plsc.subcore_barrier()
    pltpu.sync_copy(acc_sh.at[pl.ds(s * PT, PT)], out.at[wid])


def _stage1_body(d0, d1, x0, x1, wv, dinv_o, a0_o, ua1_o, u2_o):
    deg = d0[...] + d1[...]
    pos = deg > 0.0
    safe = jnp.where(pos, deg, 1.0)
    dinv = jnp.where(pos, lax.rsqrt(safe), 0.0)
    dinv_o[...] = dinv
    a0_o[...] = x0[...] * wv[0] + x1[...] * wv[1]
    ua1_o[...] = dinv * (x0[...] * wv[2] + x1[...] * wv[3])
    u2_o[...] = dinv * (x0[...] * wv[4] + x1[...] * wv[5])


def _stage2_body(t0, t1, dinv, ua1, um_o):
    dv = dinv[...]
    um_o[...] = ua1[...] + dv * dv * (t0[...] + t1[...])


def _stage3_body(s0, s1, dinv, a0, wv, out_o):
    out_o[...] = jnp.maximum(
        a0[...] + dinv[...] * (s0[...] + s1[...]) + wv[6], 0.0)


_vspec = pl.BlockSpec(memory_space=pltpu.VMEM)
_sspec = pl.BlockSpec(memory_space=pltpu.SMEM)
_nshape = jax.ShapeDtypeStruct((ROWS, LANES), jnp.float32)

_stage1 = pl.pallas_call(
    _stage1_body,
    out_shape=(_nshape, _nshape, _nshape, _nshape),
    in_specs=[_vspec, _vspec, _vspec, _vspec, _sspec],
    out_specs=(_vspec, _vspec, _vspec, _vspec),
)

_stage2 = pl.pallas_call(
    _stage2_body,
    out_shape=_nshape,
    in_specs=[_vspec, _vspec, _vspec, _vspec],
    out_specs=_vspec,
)

_stage3 = pl.pallas_call(
    _stage3_body,
    out_shape=_nshape,
    in_specs=[_vspec, _vspec, _vspec, _vspec, _sspec],
    out_specs=_vspec,
)


def _halves(parts):
    p = parts.reshape(NC, ROWS, LANES)
    return p[0], p[1]


def kernel(x, edge_index, edge_weight, W0, W1, W2, b):
    ei = edge_index.reshape(2 * E)

    pad = NP - N
    x0 = jnp.pad(x[:, 0], (0, pad)).reshape(ROWS, LANES)
    x1 = jnp.pad(x[:, 1], (0, pad)).reshape(ROWS, LANES)
    wv = jnp.stack([W0[0, 0], W0[1, 0], W1[0, 0], W1[1, 0],
                    W2[0, 0], W2[1, 0], b[0], b[0]])

    deg_parts = _degree(ei, edge_weight)
    d0, d1 = _halves(deg_parts)

    dinv, a0, ua1, u2 = _stage1(d0, d1, x0, x1, wv)

    t_parts = _prop(ei, edge_weight, u2.reshape(NP))
    t0, t1 = _halves(t_parts)
    um = _stage2(t0, t1, dinv, ua1)

    s_parts = _prop(ei, edge_weight, um.reshape(NP))
    s0, s1 = _halves(s_parts)
    out = _stage3(s0, s1, dinv, a0, wv)

    return out.reshape(NP, 1)[:N]


# speedup vs baseline: 1.1601x; 1.0379x over previous
import functools

import jax
import jax.numpy as jnp
from jax import lax
from jax.experimental import pallas as pl
from jax.experimental.pallas import tpu as pltpu
from jax.experimental.pallas import tpu_sc as plsc

N = 100000
E = 6400000
LANES = 128
ROWS = 784
NP = ROWS * LANES
NC = 2
NS = 16
NW = NC * NS
PT = NP // NS
CA = 5120
CB = 1280
NCH_A = E // CA
NCH_B = E // CB
ZCH = 1568
L = 16
RING = 4

_mesh = plsc.VectorSubcoreMesh(core_axis_name="c", subcore_axis_name="s")
_sc_params = pltpu.CompilerParams(needs_layout_passes=False)


def _zero_shared(buf, acc_sh, s):
    def fill(i, carry):
        buf[pl.ds(i * L, L)] = jnp.zeros((L,), jnp.float32)
        return carry

    lax.fori_loop(0, ZCH // L, fill, 0)
    for q in range(4):
        pltpu.sync_copy(buf.at[pl.ds(0, ZCH)],
                        acc_sh.at[pl.ds(s * PT + q * ZCH, ZCH)])


@functools.partial(
    pl.kernel,
    out_type=jax.ShapeDtypeStruct((NW, PT), jnp.float32),
    mesh=_mesh,
    compiler_params=_sc_params,
    scratch_types=[
        [pltpu.VMEM((2, CA), jnp.int32) for _ in range(RING)],
        [pltpu.VMEM((CA,), jnp.float32) for _ in range(RING)],
        [pltpu.VMEM((CA,), jnp.int32) for _ in range(RING)],
        pltpu.VMEM_SHARED((NP,), jnp.float32),
        [pltpu.SemaphoreType.DMA for _ in range(RING)],
        [pltpu.SemaphoreType.DMA for _ in range(RING)],
    ],
)
def _degree(ei_hbm, w_hbm, out, e_b, w_b, col_c, acc_sh, in_sems, sc_sems):
    c = lax.axis_index("c")
    s = lax.axis_index("s")
    wid = c * NS + s
    nch_w = jnp.where(wid < NCH_A % NW, NCH_A // NW + 1, NCH_A // NW)

    def in_start(k, r):
        base = pl.multiple_of((wid + NW * k) * CA, LANES)
        pltpu.async_copy(ei_hbm.at[pl.ds(0, 2), pl.ds(base, CA)],
                         e_b[r], in_sems[r])
        pltpu.async_copy(w_hbm.at[pl.ds(base, CA)], w_b[r], in_sems[r])

    def in_wait(r):
        pltpu.make_async_copy(ei_hbm.at[pl.ds(0, 2), pl.ds(0, CA)],
                              e_b[r], in_sems[r]).wait()
        pltpu.make_async_copy(w_hbm.at[pl.ds(0, CA)], w_b[r], in_sems[r]).wait()

    def sc_wait(r):
        pltpu.make_async_copy(w_b[r], acc_sh.at[col_c[r]], sc_sems[r]).wait()

    in_start(0, 0)
    in_start(1, 1)
    _zero_shared(w_b[3], acc_sh, s)
    plsc.subcore_barrier()

    def step(t, carry):
        for u in range(RING):
            k = t * RING + u

            @pl.when(k < nch_w)
            def _():
                in_wait(u)

                @plsc.parallel_loop(0, CA // L, unroll=8)
                def cpy(j):
                    sl = pl.ds(j * L, L)
                    col_c[u][sl] = e_b[u][1, sl]

                pltpu.async_copy(w_b[u], acc_sh.at[col_c[u]],
                                 sc_sems[u], add=True)
                nxt = (u + 2) % RING

                @pl.when(k >= 2)
                def _():
                    sc_wait(nxt)

                @pl.when(k + 2 < nch_w)
                def _():
                    in_start(k + 2, nxt)

        return carry

    lax.fori_loop(0, (NCH_A // NW + 1 + RING - 1) // RING, step, 0)
    for u in range(RING):
        @pl.when((nch_w - 1 - u) % RING <= 1)
        def _():
            sc_wait(u)
    plsc.subcore_barrier()
    pltpu.sync_copy(acc_sh.at[pl.ds(s * PT, PT)], out.at[wid])


@functools.partial(
    pl.kernel,
    out_type=jax.ShapeDtypeStruct((NW, PT), jnp.float32),
    mesh=_mesh,
    compiler_params=_sc_params,
    scratch_types=[
        pltpu.VMEM((NP,), jnp.float32),
        [pltpu.VMEM((2, CB), jnp.int32) for _ in range(RING)],
        [pltpu.VMEM((CB,), jnp.float32) for _ in range(RING)],
        [pltpu.VMEM((CB,), jnp.int32) for _ in range(RING)],
        pltpu.VMEM_SHARED((NP,), jnp.float32),
        [pltpu.SemaphoreType.DMA for _ in range(RING)],
        [pltpu.SemaphoreType.DMA for _ in range(RING)],
    ],
)
def _prop(ei_hbm, w_hbm, table_hbm, out,
          table_v, e_b, w_b, col_c, acc_sh, in_sems, sc_sems):
    c = lax.axis_index("c")
    s = lax.axis_index("s")
    wid = c * NS + s
    nch_w = jnp.where(wid < NCH_B % NW, NCH_B // NW + 1, NCH_B // NW)

    def in_start(k, r):
        base = pl.multiple_of((wid + NW * k) * CB, LANES)
        pltpu.async_copy(ei_hbm.at[pl.ds(0, 2), pl.ds(base, CB)],
                         e_b[r], in_sems[r])
        pltpu.async_copy(w_hbm.at[pl.ds(base, CB)], w_b[r], in_sems[r])

    def in_wait(r):
        pltpu.make_async_copy(ei_hbm.at[pl.ds(0, 2), pl.ds(0, CB)],
                              e_b[r], in_sems[r]).wait()
        pltpu.make_async_copy(w_hbm.at[pl.ds(0, CB)], w_b[r], in_sems[r]).wait()

    def sc_wait(r):
        pltpu.make_async_copy(w_b[r], acc_sh.at[col_c[r]], sc_sems[r]).wait()

    in_start(0, 0)
    in_start(1, 1)
    pltpu.sync_copy(table_hbm, table_v)
    _zero_shared(w_b[3], acc_sh, s)
    plsc.subcore_barrier()

    def step(t, carry):
        for u in range(RING):
            k = t * RING + u

            @pl.when(k < nch_w)
            def _():
                in_wait(u)

                @plsc.parallel_loop(0, CB // L, unroll=5)
                def grp(j):
                    sl = pl.ds(j * L, L)
                    g = plsc.load_gather(table_v, [e_b[u][0, sl]])
                    w_b[u][sl] = g * w_b[u][sl]
                    col_c[u][sl] = e_b[u][1, sl]

                pltpu.async_copy(w_b[u], acc_sh.at[col_c[u]],
                                 sc_sems[u], add=True)
                nxt = (u + 2) % RING

                @pl.when(k >= 2)
                def _():
                    sc_wait(nxt)

                @pl.when(k + 2 < nch_w)
                def _():
                    in_start(k + 2, nxt)

        return carry

    lax.fori_loop(0, (NCH_B // NW + 1 + RING - 1) // RING, step, 0)
    for u in range(RING):
        @pl.when((nch_w - 1 - u) % RING <= 1)
        def _():
            sc_wait(u)
    plsc.subcore_barrier()
    pltpu.sync_copy(acc_sh.at[pl.ds(s * PT, PT)], out.at[wid])


def _stage1_body(d0, d1, x0, x1, wv, dinv_o, a0_o, ua1_o, u2_o):
    deg = d0[...] + d1[...]
    pos = deg > 0.0
    safe = jnp.where(pos, deg, 1.0)
    dinv = jnp.where(pos, lax.rsqrt(safe), 0.0)
    dinv_o[...] = dinv
    a0_o[...] = x0[...] * wv[0] + x1[...] * wv[1]
    ua1_o[...] = dinv * (x0[...] * wv[2] + x1[...] * wv[3])
    u2_o[...] = dinv * (x0[...] * wv[4] + x1[...] * wv[5])


def _stage2_body(t0, t1, dinv, ua1, um_o):
    dv = dinv[...]
    um_o[...] = ua1[...] + dv * dv * (t0[...] + t1[...])


def _stage3_body(s0, s1, dinv, a0, wv, out_o):
    out_o[...] = jnp.maximum(
        a0[...] + dinv[...] * (s0[...] + s1[...]) + wv[6], 0.0)


_vspec = pl.BlockSpec(memory_space=pltpu.VMEM)
_sspec = pl.BlockSpec(memory_space=pltpu.SMEM)
_nshape = jax.ShapeDtypeStruct((ROWS, LANES), jnp.float32)

_stage1 = pl.pallas_call(
    _stage1_body,
    out_shape=(_nshape, _nshape, _nshape, _nshape),
    in_specs=[_vspec, _vspec, _vspec, _vspec, _sspec],
    out_specs=(_vspec, _vspec, _vspec, _vspec),
)

_stage2 = pl.pallas_call(
    _stage2_body,
    out_shape=_nshape,
    in_specs=[_vspec, _vspec, _vspec, _vspec],
    out_specs=_vspec,
)

_stage3 = pl.pallas_call(
    _stage3_body,
    out_shape=_nshape,
    in_specs=[_vspec, _vspec, _vspec, _vspec, _sspec],
    out_specs=_vspec,
)


def _halves(parts):
    p = parts.reshape(NC, ROWS, LANES)
    return p[0], p[1]


def kernel(x, edge_index, edge_weight, W0, W1, W2, b):
    pad = NP - N
    x0 = jnp.pad(x[:, 0], (0, pad)).reshape(ROWS, LANES)
    x1 = jnp.pad(x[:, 1], (0, pad)).reshape(ROWS, LANES)
    wv = jnp.stack([W0[0, 0], W0[1, 0], W1[0, 0], W1[1, 0],
                    W2[0, 0], W2[1, 0], b[0], b[0]])

    deg_parts = _degree(edge_index, edge_weight)
    d0, d1 = _halves(deg_parts)

    dinv, a0, ua1, u2 = _stage1(d0, d1, x0, x1, wv)

    t_parts = _prop(edge_index, edge_weight, u2.reshape(NP))
    t0, t1 = _halves(t_parts)
    um = _stage2(t0, t1, dinv, ua1)

    s_parts = _prop(edge_index, edge_weight, um.reshape(NP))
    s0, s1 = _halves(s_parts)
    out = _stage3(s0, s1, dinv, a0, wv)

    return out.reshape(NP, 1)[:N]


# speedup vs baseline: 1.1680x; 1.0068x over previous
import functools

import jax
import jax.numpy as jnp
from jax import lax
from jax.experimental import pallas as pl
from jax.experimental.pallas import tpu as pltpu
from jax.experimental.pallas import tpu_sc as plsc

N = 100000
E = 6400000
LANES = 128
ROWS = 784
NP = ROWS * LANES
NC = 2
NS = 16
NW = NC * NS
PT = NP // NS
CA = 5120
CB = 1280
NCH_A = E // CA
NCH_B = E // CB
ZCH = 1568
L = 16
RING = 4

_mesh = plsc.VectorSubcoreMesh(core_axis_name="c", subcore_axis_name="s")
_sc_params = pltpu.CompilerParams(needs_layout_passes=False)


def _zero_shared(buf, acc_sh, s):
    def fill(i, carry):
        buf[pl.ds(i * L, L)] = jnp.zeros((L,), jnp.float32)
        return carry

    lax.fori_loop(0, ZCH // L, fill, 0)
    for q in range(4):
        pltpu.sync_copy(buf.at[pl.ds(0, ZCH)],
                        acc_sh.at[pl.ds(s * PT + q * ZCH, ZCH)])


@functools.partial(
    pl.kernel,
    out_type=jax.ShapeDtypeStruct((NW, PT), jnp.float32),
    mesh=_mesh,
    compiler_params=_sc_params,
    scratch_types=[
        [pltpu.VMEM((2, CA), jnp.int32) for _ in range(RING)],
        [pltpu.VMEM((CA,), jnp.float32) for _ in range(RING)],
        [pltpu.VMEM((CA,), jnp.int32) for _ in range(RING)],
        pltpu.VMEM_SHARED((NP,), jnp.float32),
        [pltpu.SemaphoreType.DMA for _ in range(RING)],
        [pltpu.SemaphoreType.DMA for _ in range(RING)],
    ],
)
def _degree(ei_hbm, w_hbm, out, e_b, w_b, col_c, acc_sh, in_sems, sc_sems):
    c = lax.axis_index("c")
    s = lax.axis_index("s")
    wid = c * NS + s
    nch_w = jnp.where(wid < NCH_A % NW, NCH_A // NW + 1, NCH_A // NW)

    def in_start(k, r):
        base = pl.multiple_of((wid + NW * k) * CA, LANES)
        pltpu.async_copy(ei_hbm.at[pl.ds(0, 2), pl.ds(base, CA)],
                         e_b[r], in_sems[r])
        pltpu.async_copy(w_hbm.at[pl.ds(base, CA)], w_b[r], in_sems[r])

    def in_wait(r):
        pltpu.make_async_copy(ei_hbm.at[pl.ds(0, 2), pl.ds(0, CA)],
                              e_b[r], in_sems[r]).wait()
        pltpu.make_async_copy(w_hbm.at[pl.ds(0, CA)], w_b[r], in_sems[r]).wait()

    def sc_wait(r):
        pltpu.make_async_copy(w_b[r], acc_sh.at[col_c[r]], sc_sems[r]).wait()

    in_start(0, 0)
    in_start(1, 1)
    _zero_shared(w_b[3], acc_sh, s)
    plsc.subcore_barrier()

    def step(t, carry):
        for u in range(RING):
            k = t * RING + u

            @pl.when(k < nch_w)
            def _():
                in_wait(u)

                @plsc.parallel_loop(0, CA // L, unroll=8)
                def cpy(j):
                    sl = pl.ds(j * L, L)
                    col_c[u][sl] = e_b[u][1, sl]

                pltpu.async_copy(w_b[u], acc_sh.at[col_c[u]],
                                 sc_sems[u], add=True)
                nxt = (u + 2) % RING

                @pl.when(k >= 2)
                def _():
                    sc_wait(nxt)

                @pl.when(k + 2 < nch_w)
                def _():
                    in_start(k + 2, nxt)

        return carry

    lax.fori_loop(0, (NCH_A // NW + 1 + RING - 1) // RING, step, 0)
    for u in range(RING):
        @pl.when((nch_w - 1 - u) % RING <= 1)
        def _():
            sc_wait(u)
    plsc.subcore_barrier()
    pltpu.sync_copy(acc_sh.at[pl.ds(s * PT, PT)], out.at[wid])


@functools.partial(
    pl.kernel,
    out_type=jax.ShapeDtypeStruct((NW, PT), jnp.float32),
    mesh=_mesh,
    compiler_params=_sc_params,
    scratch_types=[
        pltpu.VMEM((NP,), jnp.float32),
        [pltpu.VMEM((2, CB), jnp.int32) for _ in range(RING)],
        [pltpu.VMEM((CB,), jnp.float32) for _ in range(RING)],
        [pltpu.VMEM((CB,), jnp.int32) for _ in range(RING)],
        pltpu.VMEM_SHARED((NP,), jnp.float32),
        [pltpu.SemaphoreType.DMA for _ in range(RING)],
        [pltpu.SemaphoreType.DMA for _ in range(RING)],
    ],
)
def _prop(ei_hbm, w_hbm, table_hbm, out,
          table_v, e_b, w_b, col_c, acc_sh, in_sems, sc_sems):
    c = lax.axis_index("c")
    s = lax.axis_index("s")
    wid = c * NS + s
    nch_w = jnp.where(wid < NCH_B % NW, NCH_B // NW + 1, NCH_B // NW)

    def in_start(k, r):
        base = pl.multiple_of((wid + NW * k) * CB, LANES)
        pltpu.async_copy(ei_hbm.at[pl.ds(0, 2), pl.ds(base, CB)],
                         e_b[r], in_sems[r])
        pltpu.async_copy(w_hbm.at[pl.ds(base, CB)], w_b[r], in_sems[r])

    def in_wait(r):
        pltpu.make_async_copy(ei_hbm.at[pl.ds(0, 2), pl.ds(0, CB)],
                              e_b[r], in_sems[r]).wait()
        pltpu.make_async_copy(w_hbm.at[pl.ds(0, CB)], w_b[r], in_sems[r]).wait()

    def sc_wait(r):
        pltpu.make_async_copy(w_b[r], acc_sh.at[col_c[r]], sc_sems[r]).wait()

    in_start(0, 0)
    in_start(1, 1)
    pltpu.sync_copy(table_hbm, table_v)
    _zero_shared(w_b[3], acc_sh, s)
    plsc.subcore_barrier()

    def step(t, carry):
        for u in range(RING):
            k = t * RING + u

            @pl.when(k < nch_w)
            def _():
                in_wait(u)

                @plsc.parallel_loop(0, CB // L, unroll=8)
                def grp(j):
                    sl = pl.ds(j * L, L)
                    g = plsc.load_gather(table_v, [e_b[u][0, sl]])
                    w_b[u][sl] = g * w_b[u][sl]
                    col_c[u][sl] = e_b[u][1, sl]

                pltpu.async_copy(w_b[u], acc_sh.at[col_c[u]],
                                 sc_sems[u], add=True)
                nxt = (u + 2) % RING

                @pl.when(k >= 2)
                def _():
                    sc_wait(nxt)

                @pl.when(k + 2 < nch_w)
                def _():
                    in_start(k + 2, nxt)

        return carry

    lax.fori_loop(0, (NCH_B // NW + 1 + RING - 1) // RING, step, 0)
    for u in range(RING):
        @pl.when((nch_w - 1 - u) % RING <= 1)
        def _():
            sc_wait(u)
    plsc.subcore_barrier()
    pltpu.sync_copy(acc_sh.at[pl.ds(s * PT, PT)], out.at[wid])


def _stage1_body(d0, d1, x0, x1, wv, dinv_o, a0_o, ua1_o, u2_o):
    deg = d0[...] + d1[...]
    pos = deg > 0.0
    safe = jnp.where(pos, deg, 1.0)
    dinv = jnp.where(pos, lax.rsqrt(safe), 0.0)
    dinv_o[...] = dinv
    a0_o[...] = x0[...] * wv[0] + x1[...] * wv[1]
    ua1_o[...] = dinv * (x0[...] * wv[2] + x1[...] * wv[3])
    u2_o[...] = dinv * (x0[...] * wv[4] + x1[...] * wv[5])


def _stage2_body(t0, t1, dinv, ua1, um_o):
    dv = dinv[...]
    um_o[...] = ua1[...] + dv * dv * (t0[...] + t1[...])


def _stage3_body(s0, s1, dinv, a0, wv, out_o):
    out_o[...] = jnp.maximum(
        a0[...] + dinv[...] * (s0[...] + s1[...]) + wv[6], 0.0)


_vspec = pl.BlockSpec(memory_space=pltpu.VMEM)
_sspec = pl.BlockSpec(memory_space=pltpu.SMEM)
_nshape = jax.ShapeDtypeStruct((ROWS, LANES), jnp.float32)

_stage1 = pl.pallas_call(
    _stage1_body,
    out_shape=(_nshape, _nshape, _nshape, _nshape),
    in_specs=[_vspec, _vspec, _vspec, _vspec, _sspec],
    out_specs=(_vspec, _vspec, _vspec, _vspec),
)

_stage2 = pl.pallas_call(
    _stage2_body,
    out_shape=_nshape,
    in_specs=[_vspec, _vspec, _vspec, _vspec],
    out_specs=_vspec,
)

_stage3 = pl.pallas_call(
    _stage3_body,
    out_shape=_nshape,
    in_specs=[_vspec, _vspec, _vspec, _vspec, _sspec],
    out_specs=_vspec,
)


def _halves(parts):
    p = parts.reshape(NC, ROWS, LANES)
    return p[0], p[1]


def kernel(x, edge_index, edge_weight, W0, W1, W2, b):
    pad = NP - N
    x0 = jnp.pad(x[:, 0], (0, pad)).reshape(ROWS, LANES)
    x1 = jnp.pad(x[:, 1], (0, pad)).reshape(ROWS, LANES)
    wv = jnp.stack([W0[0, 0], W0[1, 0], W1[0, 0], W1[1, 0],
                    W2[0, 0], W2[1, 0], b[0], b[0]])

    deg_parts = _degree(edge_index, edge_weight)
    d0, d1 = _halves(deg_parts)

    dinv, a0, ua1, u2 = _stage1(d0, d1, x0, x1, wv)

    t_parts = _prop(edge_index, edge_weight, u2.reshape(NP))
    t0, t1 = _halves(t_parts)
    um = _stage2(t0, t1, dinv, ua1)

    s_parts = _prop(edge_index, edge_weight, um.reshape(NP))
    s0, s1 = _halves(s_parts)
    out = _stage3(s0, s1, dinv, a0, wv)

    return out.reshape(NP, 1)[:N]
